# baseline (speedup 1.0000x reference)
import functools

import jax
import jax.numpy as jnp
from jax import lax
from jax.experimental import pallas as pl
from jax.experimental.pallas import tpu as pltpu
from jax.experimental.pallas import tpu_sc as plsc

B, N, M, K, IN_CH = 4, 8192, 1024, 32, 16
CH0 = IN_CH + 3
C1, C2, C3 = 32, 32, 64
R = B * K * M
NTOT = float(B * M * K)

F32 = jnp.float32
I32 = jnp.int32


_NS = 8


def _fps_body(xs_ref, ys_ref, zs_ref, cx_ref, cy_ref, cz_ref, dist_ref):
    sh = (B, _NS, N // _NS)
    lane = (lax.broadcasted_iota(I32, sh, 1) * (N // _NS)
            + lax.broadcasted_iota(I32, sh, 2))
    dist_ref[...] = jnp.full(sh, 1e10, F32)

    def body(i, far):
        xs = xs_ref[...]
        ys = ys_ref[...]
        zs = zs_ref[...]
        msk = lane == far[:, None, None]
        cx = jnp.sum(jnp.where(msk, xs, 0.0), axis=(1, 2))
        cy = jnp.sum(jnp.where(msk, ys, 0.0), axis=(1, 2))
        cz = jnp.sum(jnp.where(msk, zs, 0.0), axis=(1, 2))
        cx_ref[pl.ds(i, 1), :] = cx[None, :]
        cy_ref[pl.ds(i, 1), :] = cy[None, :]
        cz_ref[pl.ds(i, 1), :] = cz[None, :]
        dx = xs - cx[:, None, None]
        dy = ys - cy[:, None, None]
        dz = zs - cz[:, None, None]
        d = dx * dx + (dy * dy + dz * dz)
        dist = jnp.minimum(dist_ref[...], d)
        dist_ref[...] = dist
        mx = jnp.max(dist, axis=(1, 2), keepdims=True)
        far2 = jnp.min(jnp.where(dist == mx, lane, N), axis=(1, 2))
        return far2.astype(I32)

    lax.fori_loop(0, M, body, jnp.zeros((B,), I32))


_fps_call = pl.pallas_call(
    _fps_body,
    grid=(1,),
    in_specs=[pl.BlockSpec((B, _NS, N // _NS), lambda i: (0, 0, 0))] * 3,
    out_specs=[pl.BlockSpec((M, B), lambda i: (0, 0))] * 3,
    out_shape=[jax.ShapeDtypeStruct((M, B), F32)] * 3,
    scratch_shapes=[pltpu.VMEM((B, _NS, N // _NS), F32)],
)


MT = 128


def _knn_body(qx_ref, qy_ref, qz_ref, xs_ref, ys_ref, zs_ref, nn_ref, d2_ref):
    b = pl.program_id(0)
    qx = qx_ref[0, 0, 0, :]
    qy = qy_ref[0, 0, 0, :]
    qz = qz_ref[0, 0, 0, :]
    xs = xs_ref[0, 0, :]
    ys = ys_ref[0, 0, :]
    zs = zs_ref[0, 0, :]
    qq = qx * qx + qy * qy + qz * qz
    xx = xs * xs + ys * ys + zs * zs
    qxb = qx.astype(jnp.bfloat16).astype(F32)
    qyb = qy.astype(jnp.bfloat16).astype(F32)
    qzb = qz.astype(jnp.bfloat16).astype(F32)
    xsb = xs.astype(jnp.bfloat16).astype(F32)
    ysb = ys.astype(jnp.bfloat16).astype(F32)
    zsb = zs.astype(jnp.bfloat16).astype(F32)
    dot = (qxb[:, None] * xsb[None, :] + qyb[:, None] * ysb[None, :]
           + qzb[:, None] * zsb[None, :])
    d2 = qq[:, None] + xx[None, :] - 2.0 * dot
    d2_ref[...] = d2
    lane = lax.broadcasted_iota(I32, (MT, N), 1)
    m0 = jnp.min(d2, axis=1, keepdims=True)

    def body(k, m):
        d2 = d2_ref[...]
        idx = jnp.min(jnp.where(d2 == m, lane, N), axis=1, keepdims=True)
        nn_ref[0, pl.ds(k, 1), :] = jnp.reshape(idx, (1, MT)) + b * N
        d2m = jnp.where(lane == idx, jnp.inf, d2)
        d2_ref[...] = d2m
        return jnp.min(d2m, axis=1, keepdims=True)

    lax.fori_loop(0, K, body, m0)


_knn_call = pl.pallas_call(
    _knn_body,
    grid=(B, M // MT),
    in_specs=[pl.BlockSpec((1, 1, 1, MT), lambda b, t: (b, t, 0, 0))] * 3
    + [pl.BlockSpec((1, 1, N), lambda b, t: (b, 0, 0))] * 3,
    out_specs=pl.BlockSpec((1, K, MT), lambda b, t: (b, 0, t)),
    out_shape=jax.ShapeDtypeStruct((B, K, M), I32),
    scratch_shapes=[pltpu.VMEM((MT, N), F32)],
)


def _pre_body(x_ref, q_ref, w_ref, a_ref, c_ref):
    a_ref[0] = jnp.dot(x_ref[0], w_ref[...], preferred_element_type=F32)
    c_ref[0] = jnp.dot(q_ref[0], w_ref[0:8, :], preferred_element_type=F32)


_pre_call = pl.pallas_call(
    _pre_body,
    grid=(B,),
    in_specs=[
        pl.BlockSpec((1, N, 24), lambda b: (b, 0, 0)),
        pl.BlockSpec((1, M, 8), lambda b: (b, 0, 0)),
        pl.BlockSpec((24, C1), lambda b: (0, 0)),
    ],
    out_specs=[
        pl.BlockSpec((1, N, C1), lambda b: (b, 0, 0)),
        pl.BlockSpec((1, M, C1), lambda b: (b, 0, 0)),
    ],
    out_shape=[
        jax.ShapeDtypeStruct((B, N, C1), F32),
        jax.ShapeDtypeStruct((B, M, C1), F32),
    ],
)


_SC_CH = 128
_SC_NW = 32
_SC_PER_W = R // _SC_NW


@functools.cache
def _get_sc_gather():
    mesh = plsc.VectorSubcoreMesh(core_axis_name="c", subcore_axis_name="s")

    @functools.partial(
        pl.kernel,
        out_type=jax.ShapeDtypeStruct((R, C1), F32),
        mesh=mesh,
        scratch_types=[
            pltpu.VMEM((_SC_CH,), I32),
            pltpu.VMEM((_SC_CH, C1), F32),
            pltpu.SemaphoreType.DMA,
        ],
        compiler_params=pltpu.CompilerParams(use_tc_tiling_on_sc=False),
    )
    def _sc_gather(table_hbm, idx_hbm, out_hbm, idx_v, rows_v, sem):
        wid = lax.axis_index("s") * 2 + lax.axis_index("c")
        base = wid * _SC_PER_W

        def chunk(i, carry):
            off = pl.multiple_of(base + i * _SC_CH, _SC_CH)
            pltpu.sync_copy(idx_hbm.at[pl.ds(off, _SC_CH)], idx_v)
            pltpu.async_copy(table_hbm.at[idx_v], rows_v, sem).wait()
            pltpu.sync_copy(rows_v, out_hbm.at[pl.ds(off, _SC_CH)])
            return carry

        lax.fori_loop(0, _SC_PER_W // _SC_CH, chunk, 0)

    return _sc_gather


_GB = 8


def _stats_body(g_ref, c_ref, s_ref, q_ref):
    h = g_ref[...] - c_ref[...]
    ps = jnp.broadcast_to(jnp.sum(h, axis=(0, 1))[None, :], (8, C1))
    pq = jnp.broadcast_to(jnp.sum(h * h, axis=(0, 1))[None, :], (8, C1))

    @pl.when(pl.program_id(0) == 0)
    def _():
        s_ref[...] = jnp.zeros((8, C1), F32)
        q_ref[...] = jnp.zeros((8, C1), F32)

    s_ref[...] += ps
    q_ref[...] += pq


_stats_call = pl.pallas_call(
    _stats_body,
    grid=(B * K // _GB,),
    in_specs=[
        pl.BlockSpec((_GB, M, C1), lambda i: (i, 0, 0)),
        pl.BlockSpec((1, M, C1), lambda i: (i // (K // _GB), 0, 0)),
    ],
    out_specs=[pl.BlockSpec((8, C1), lambda i: (0, 0))] * 2,
    out_shape=[jax.ShapeDtypeStruct((8, C1), F32)] * 2,
)


def _bn_gelu(x, s_ref, q_ref, gamma_ref, beta_ref, ch):
    mean = s_ref[0, :] / NTOT
    var = q_ref[0, :] / NTOT - mean * mean
    inv = gamma_ref[0, :] / jnp.sqrt(var + 1e-5)
    x = (x - mean[None, :]) * inv[None, :] + beta_ref[0, :][None, :]
    return 0.5 * x * (1.0 + lax.erf(x * 0.7071067811865476))


def _layer_body(sub_c, chin, chout, *refs):
    if sub_c:
        g_ref, c_ref, s_ref, q_ref, gm_ref, bt_ref, w_ref, y_ref, s2_ref, q2_ref = refs
        h = g_ref[...] - c_ref[...]
    else:
        g_ref, s_ref, q_ref, gm_ref, bt_ref, w_ref, y_ref, s2_ref, q2_ref = refs
        h = g_ref[...]
    x = _bn_gelu(h, s_ref, q_ref, gm_ref, bt_ref, chin)
    y = jnp.dot(x.reshape(_GB * M, chin), w_ref[...],
                preferred_element_type=F32)
    y_ref[...] = y.reshape(_GB, M, chout)
    ps = jnp.broadcast_to(jnp.sum(y, axis=0)[None, :], (8, chout))
    pq = jnp.broadcast_to(jnp.sum(y * y, axis=0)[None, :], (8, chout))

    @pl.when(pl.program_id(0) == 0)
    def _():
        s2_ref[...] = jnp.zeros((8, chout), F32)
        q2_ref[...] = jnp.zeros((8, chout), F32)

    s2_ref[...] += ps
    q2_ref[...] += pq


def _make_layer_call(sub_c, chin, chout):
    in_specs = [pl.BlockSpec((_GB, M, chin), lambda i: (i, 0, 0))]
    if sub_c:
        in_specs.append(
            pl.BlockSpec((1, M, chin), lambda i: (i // (K // _GB), 0, 0)))
    in_specs += [
        pl.BlockSpec((8, chin), lambda i: (0, 0)),
        pl.BlockSpec((8, chin), lambda i: (0, 0)),
        pl.BlockSpec((1, chin), lambda i: (0, 0)),
        pl.BlockSpec((1, chin), lambda i: (0, 0)),
        pl.BlockSpec((chin, chout), lambda i: (0, 0)),
    ]
    return pl.pallas_call(
        functools.partial(_layer_body, sub_c, chin, chout),
        grid=(B * K // _GB,),
        in_specs=in_specs,
        out_specs=[
            pl.BlockSpec((_GB, M, chout), lambda i: (i, 0, 0)),
            pl.BlockSpec((8, chout), lambda i: (0, 0)),
            pl.BlockSpec((8, chout), lambda i: (0, 0)),
        ],
        out_shape=[
            jax.ShapeDtypeStruct((B * K, M, chout), F32),
            jax.ShapeDtypeStruct((8, chout), F32),
            jax.ShapeDtypeStruct((8, chout), F32),
        ],
    )


_layer2_call = _make_layer_call(True, C1, C2)
_layer3_call = _make_layer_call(False, C2, C3)


QT = 256


def _pool_body(y_ref, s_ref, q_ref, gm_ref, bt_ref, o_ref):
    x = _bn_gelu(y_ref[0], s_ref, q_ref, gm_ref, bt_ref, C3)
    o_ref[0] = jnp.max(x, axis=0)


_pool_call = pl.pallas_call(
    _pool_body,
    grid=(B, M // QT),
    in_specs=[
        pl.BlockSpec((1, K, QT, C3), lambda b, t: (b, 0, t, 0)),
        pl.BlockSpec((8, C3), lambda b, t: (0, 0)),
        pl.BlockSpec((8, C3), lambda b, t: (0, 0)),
        pl.BlockSpec((1, C3), lambda b, t: (0, 0)),
        pl.BlockSpec((1, C3), lambda b, t: (0, 0)),
    ],
    out_specs=pl.BlockSpec((1, QT, C3), lambda b, t: (b, t, 0)),
    out_shape=jax.ShapeDtypeStruct((B, M, C3), F32),
)


@jax.jit
def kernel(xyz, feat, W0, gamma0, beta0, W1, gamma1, beta1, W2, gamma2, beta2):
    xs = xyz[..., 0]
    ys = xyz[..., 1]
    zs = xyz[..., 2]
    fsh = (B, _NS, N // _NS)
    cxs, cys, czs = _fps_call(xs.reshape(fsh), ys.reshape(fsh), zs.reshape(fsh))
    new_xyz = jnp.stack([cxs, cys, czs], axis=-1).transpose(1, 0, 2)
    qshape = (B, M // MT, 1, MT)
    nn = _knn_call(
        cxs.T.reshape(qshape), cys.T.reshape(qshape), czs.T.reshape(qshape),
        xs.reshape(B, 1, N), ys.reshape(B, 1, N), zs.reshape(B, 1, N))

    Xp = jnp.pad(jnp.concatenate([xyz, feat], axis=-1), ((0, 0), (0, 0), (0, 5)))
    Qp = jnp.pad(new_xyz, ((0, 0), (0, 0), (0, 5)))
    W0p = jnp.pad(W0, ((0, 5), (0, 0)))
    A, C = _pre_call(Xp, Qp, W0p)

    G = _get_sc_gather()(A.reshape(B * N, C1), nn.reshape(-1))
    Gr = G.reshape(B * K, M, C1)

    s1, q1 = _stats_call(Gr, C)
    g0 = gamma0.reshape(1, C1)
    b0 = beta0.reshape(1, C1)
    y2, s2, q2 = _layer2_call(Gr, C, s1, q1, g0, b0, W1)
    y3, s3, q3 = _layer3_call(
        y2, s2, q2, gamma1.reshape(1, C2), beta1.reshape(1, C2), W2)
    h = _pool_call(
        y3.reshape(B, K, M, C3), s3, q3,
        gamma2.reshape(1, C3), beta2.reshape(1, C3))
    return new_xyz, h


# speedup vs baseline: 1.0933x; 1.0933x over previous
import functools

import jax
import jax.numpy as jnp
from jax import lax
from jax.experimental import pallas as pl
from jax.experimental.pallas import tpu as pltpu
from jax.experimental.pallas import tpu_sc as plsc

B, N, M, K, IN_CH = 4, 8192, 1024, 32, 16
CH0 = IN_CH + 3
C1, C2, C3 = 32, 32, 64
R = B * K * M
NTOT = float(B * M * K)

F32 = jnp.float32
I32 = jnp.int32


_NS = 8


def _fps_body(xs_ref, ys_ref, zs_ref, cx_ref, cy_ref, cz_ref, dist_ref):
    sh = (B, _NS, N // _NS)
    lane = (lax.broadcasted_iota(I32, sh, 1) * (N // _NS)
            + lax.broadcasted_iota(I32, sh, 2))
    dist_ref[...] = jnp.full(sh, 1e10, F32)

    def body(i, far):
        xs = xs_ref[...]
        ys = ys_ref[...]
        zs = zs_ref[...]
        msk = lane == far[:, None, None]
        cx = jnp.sum(jnp.where(msk, xs, 0.0), axis=(1, 2))
        cy = jnp.sum(jnp.where(msk, ys, 0.0), axis=(1, 2))
        cz = jnp.sum(jnp.where(msk, zs, 0.0), axis=(1, 2))
        cx_ref[pl.ds(i, 1), :] = cx[None, :]
        cy_ref[pl.ds(i, 1), :] = cy[None, :]
        cz_ref[pl.ds(i, 1), :] = cz[None, :]
        dx = xs - cx[:, None, None]
        dy = ys - cy[:, None, None]
        dz = zs - cz[:, None, None]
        d = dx * dx + (dy * dy + dz * dz)
        dist = jnp.minimum(dist_ref[...], d)
        dist_ref[...] = dist
        mx = jnp.max(dist, axis=(1, 2), keepdims=True)
        far2 = jnp.min(jnp.where(dist == mx, lane, N), axis=(1, 2))
        return far2.astype(I32)

    lax.fori_loop(0, M, body, jnp.zeros((B,), I32))


_fps_call = pl.pallas_call(
    _fps_body,
    grid=(1,),
    in_specs=[pl.BlockSpec((B, _NS, N // _NS), lambda i: (0, 0, 0))] * 3,
    out_specs=[pl.BlockSpec((M, B), lambda i: (0, 0))] * 3,
    out_shape=[jax.ShapeDtypeStruct((M, B), F32)] * 3,
    scratch_shapes=[pltpu.VMEM((B, _NS, N // _NS), F32)],
)


MT = 256


def _knn_body(qx_ref, qy_ref, qz_ref, xs_ref, ys_ref, zs_ref, nn_ref, d2_ref):
    b = pl.program_id(0)
    qx = qx_ref[0, 0, 0, :]
    qy = qy_ref[0, 0, 0, :]
    qz = qz_ref[0, 0, 0, :]
    xs = xs_ref[0, 0, :]
    ys = ys_ref[0, 0, :]
    zs = zs_ref[0, 0, :]
    qq = qx * qx + qy * qy + qz * qz
    xx = xs * xs + ys * ys + zs * zs
    qxb = qx.astype(jnp.bfloat16).astype(F32)
    qyb = qy.astype(jnp.bfloat16).astype(F32)
    qzb = qz.astype(jnp.bfloat16).astype(F32)
    xsb = xs.astype(jnp.bfloat16).astype(F32)
    ysb = ys.astype(jnp.bfloat16).astype(F32)
    zsb = zs.astype(jnp.bfloat16).astype(F32)
    dot = (qxb[:, None] * xsb[None, :] + qyb[:, None] * ysb[None, :]
           + qzb[:, None] * zsb[None, :])
    d2_ref[...] = qq[:, None] + xx[None, :] - 2.0 * dot
    lane = lax.broadcasted_iota(I32, (MT, N), 1)

    def body(k, carry):
        d2 = d2_ref[...]
        m = jnp.min(d2, axis=1, keepdims=True)
        idx = jnp.min(jnp.where(d2 == m, lane, N), axis=1, keepdims=True)
        nn_ref[0, pl.ds(k, 1), :] = jnp.reshape(idx, (1, MT)) + b * N
        d2_ref[...] = jnp.where(lane == idx, jnp.inf, d2)
        return carry

    lax.fori_loop(0, K, body, 0)


_knn_call = pl.pallas_call(
    _knn_body,
    grid=(B, M // MT),
    in_specs=[pl.BlockSpec((1, 1, 1, MT), lambda b, t: (b, t, 0, 0))] * 3
    + [pl.BlockSpec((1, 1, N), lambda b, t: (b, 0, 0))] * 3,
    out_specs=pl.BlockSpec((1, K, MT), lambda b, t: (b, 0, t)),
    out_shape=jax.ShapeDtypeStruct((B, K, M), I32),
    scratch_shapes=[pltpu.VMEM((MT, N), F32)],
)


def _pre_body(x_ref, q_ref, w_ref, a_ref, c_ref):
    a_ref[0] = jnp.dot(x_ref[0], w_ref[...], preferred_element_type=F32)
    c_ref[0] = jnp.dot(q_ref[0], w_ref[0:8, :], preferred_element_type=F32)


_pre_call = pl.pallas_call(
    _pre_body,
    grid=(B,),
    in_specs=[
        pl.BlockSpec((1, N, 24), lambda b: (b, 0, 0)),
        pl.BlockSpec((1, M, 8), lambda b: (b, 0, 0)),
        pl.BlockSpec((24, C1), lambda b: (0, 0)),
    ],
    out_specs=[
        pl.BlockSpec((1, N, C1), lambda b: (b, 0, 0)),
        pl.BlockSpec((1, M, C1), lambda b: (b, 0, 0)),
    ],
    out_shape=[
        jax.ShapeDtypeStruct((B, N, C1), F32),
        jax.ShapeDtypeStruct((B, M, C1), F32),
    ],
)


_SC_CH = 128
_SC_NW = 32
_SC_PER_W = R // _SC_NW


@functools.cache
def _get_sc_gather():
    mesh = plsc.VectorSubcoreMesh(core_axis_name="c", subcore_axis_name="s")

    @functools.partial(
        pl.kernel,
        out_type=jax.ShapeDtypeStruct((R, C1), F32),
        mesh=mesh,
        scratch_types=[
            pltpu.VMEM((_SC_CH,), I32),
            pltpu.VMEM((_SC_CH, C1), F32),
            pltpu.SemaphoreType.DMA,
        ],
        compiler_params=pltpu.CompilerParams(use_tc_tiling_on_sc=False),
    )
    def _sc_gather(table_hbm, idx_hbm, out_hbm, idx_v, rows_v, sem):
        wid = lax.axis_index("s") * 2 + lax.axis_index("c")
        base = wid * _SC_PER_W

        def chunk(i, carry):
            off = pl.multiple_of(base + i * _SC_CH, _SC_CH)
            pltpu.sync_copy(idx_hbm.at[pl.ds(off, _SC_CH)], idx_v)
            pltpu.async_copy(table_hbm.at[idx_v], rows_v, sem).wait()
            pltpu.sync_copy(rows_v, out_hbm.at[pl.ds(off, _SC_CH)])
            return carry

        lax.fori_loop(0, _SC_PER_W // _SC_CH, chunk, 0)

    return _sc_gather


_GB = 8


def _stats_body(g_ref, c_ref, s_ref, q_ref):
    h = g_ref[...] - c_ref[...]
    ps = jnp.broadcast_to(jnp.sum(h, axis=(0, 1))[None, :], (8, C1))
    pq = jnp.broadcast_to(jnp.sum(h * h, axis=(0, 1))[None, :], (8, C1))

    @pl.when(pl.program_id(0) == 0)
    def _():
        s_ref[...] = jnp.zeros((8, C1), F32)
        q_ref[...] = jnp.zeros((8, C1), F32)

    s_ref[...] += ps
    q_ref[...] += pq


_stats_call = pl.pallas_call(
    _stats_body,
    grid=(B * K // _GB,),
    in_specs=[
        pl.BlockSpec((_GB, M, C1), lambda i: (i, 0, 0)),
        pl.BlockSpec((1, M, C1), lambda i: (i // (K // _GB), 0, 0)),
    ],
    out_specs=[pl.BlockSpec((8, C1), lambda i: (0, 0))] * 2,
    out_shape=[jax.ShapeDtypeStruct((8, C1), F32)] * 2,
)


def _bn_gelu(x, s_ref, q_ref, gamma_ref, beta_ref, ch):
    mean = s_ref[0, :] / NTOT
    var = q_ref[0, :] / NTOT - mean * mean
    inv = gamma_ref[0, :] / jnp.sqrt(var + 1e-5)
    x = (x - mean[None, :]) * inv[None, :] + beta_ref[0, :][None, :]
    return 0.5 * x * (1.0 + lax.erf(x * 0.7071067811865476))


def _layer_body(sub_c, chin, chout, *refs):
    if sub_c:
        g_ref, c_ref, s_ref, q_ref, gm_ref, bt_ref, w_ref, y_ref, s2_ref, q2_ref = refs
        h = g_ref[...] - c_ref[...]
    else:
        g_ref, s_ref, q_ref, gm_ref, bt_ref, w_ref, y_ref, s2_ref, q2_ref = refs
        h = g_ref[...]
    x = _bn_gelu(h, s_ref, q_ref, gm_ref, bt_ref, chin)
    y = jnp.dot(x.reshape(_GB * M, chin), w_ref[...],
                preferred_element_type=F32)
    y_ref[...] = y.reshape(_GB, M, chout)
    ps = jnp.broadcast_to(jnp.sum(y, axis=0)[None, :], (8, chout))
    pq = jnp.broadcast_to(jnp.sum(y * y, axis=0)[None, :], (8, chout))

    @pl.when(pl.program_id(0) == 0)
    def _():
        s2_ref[...] = jnp.zeros((8, chout), F32)
        q2_ref[...] = jnp.zeros((8, chout), F32)

    s2_ref[...] += ps
    q2_ref[...] += pq


def _make_layer_call(sub_c, chin, chout):
    in_specs = [pl.BlockSpec((_GB, M, chin), lambda i: (i, 0, 0))]
    if sub_c:
        in_specs.append(
            pl.BlockSpec((1, M, chin), lambda i: (i // (K // _GB), 0, 0)))
    in_specs += [
        pl.BlockSpec((8, chin), lambda i: (0, 0)),
        pl.BlockSpec((8, chin), lambda i: (0, 0)),
        pl.BlockSpec((1, chin), lambda i: (0, 0)),
        pl.BlockSpec((1, chin), lambda i: (0, 0)),
        pl.BlockSpec((chin, chout), lambda i: (0, 0)),
    ]
    return pl.pallas_call(
        functools.partial(_layer_body, sub_c, chin, chout),
        grid=(B * K // _GB,),
        in_specs=in_specs,
        out_specs=[
            pl.BlockSpec((_GB, M, chout), lambda i: (i, 0, 0)),
            pl.BlockSpec((8, chout), lambda i: (0, 0)),
            pl.BlockSpec((8, chout), lambda i: (0, 0)),
        ],
        out_shape=[
            jax.ShapeDtypeStruct((B * K, M, chout), F32),
            jax.ShapeDtypeStruct((8, chout), F32),
            jax.ShapeDtypeStruct((8, chout), F32),
        ],
    )


_layer2_call = _make_layer_call(True, C1, C2)
_layer3_call = _make_layer_call(False, C2, C3)


QT = 256


def _pool_body(y_ref, s_ref, q_ref, gm_ref, bt_ref, o_ref):
    x = _bn_gelu(y_ref[0], s_ref, q_ref, gm_ref, bt_ref, C3)
    o_ref[0] = jnp.max(x, axis=0)


_pool_call = pl.pallas_call(
    _pool_body,
    grid=(B, M // QT),
    in_specs=[
        pl.BlockSpec((1, K, QT, C3), lambda b, t: (b, 0, t, 0)),
        pl.BlockSpec((8, C3), lambda b, t: (0, 0)),
        pl.BlockSpec((8, C3), lambda b, t: (0, 0)),
        pl.BlockSpec((1, C3), lambda b, t: (0, 0)),
        pl.BlockSpec((1, C3), lambda b, t: (0, 0)),
    ],
    out_specs=pl.BlockSpec((1, QT, C3), lambda b, t: (b, t, 0)),
    out_shape=jax.ShapeDtypeStruct((B, M, C3), F32),
)


@jax.jit
def kernel(xyz, feat, W0, gamma0, beta0, W1, gamma1, beta1, W2, gamma2, beta2):
    xs = xyz[..., 0]
    ys = xyz[..., 1]
    zs = xyz[..., 2]
    fsh = (B, _NS, N // _NS)
    cxs, cys, czs = _fps_call(xs.reshape(fsh), ys.reshape(fsh), zs.reshape(fsh))
    new_xyz = jnp.stack([cxs, cys, czs], axis=-1).transpose(1, 0, 2)
    qshape = (B, M // MT, 1, MT)
    nn = _knn_call(
        cxs.T.reshape(qshape), cys.T.reshape(qshape), czs.T.reshape(qshape),
        xs.reshape(B, 1, N), ys.reshape(B, 1, N), zs.reshape(B, 1, N))

    Xp = jnp.pad(jnp.concatenate([xyz, feat], axis=-1), ((0, 0), (0, 0), (0, 5)))
    Qp = jnp.pad(new_xyz, ((0, 0), (0, 0), (0, 5)))
    W0p = jnp.pad(W0, ((0, 5), (0, 0)))
    A, C = _pre_call(Xp, Qp, W0p)

    G = _get_sc_gather()(A.reshape(B * N, C1), nn.reshape(-1))
    Gr = G.reshape(B * K, M, C1)

    s1, q1 = _stats_call(Gr, C)
    g0 = gamma0.reshape(1, C1)
    b0 = beta0.reshape(1, C1)
    y2, s2, q2 = _layer2_call(Gr, C, s1, q1, g0, b0, W1)
    y3, s3, q3 = _layer3_call(
        y2, s2, q2, gamma1.reshape(1, C2), beta1.reshape(1, C2), W2)
    h = _pool_call(
        y3.reshape(B, K, M, C3), s3, q3,
        gamma2.reshape(1, C3), beta2.reshape(1, C3))
    return new_xyz, h


# speedup vs baseline: 1.1332x; 1.0366x over previous
import functools

import jax
import jax.numpy as jnp
from jax import lax
from jax.experimental import pallas as pl
from jax.experimental.pallas import tpu as pltpu
from jax.experimental.pallas import tpu_sc as plsc

B, N, M, K, IN_CH = 4, 8192, 1024, 32, 16
CH0 = IN_CH + 3
C1, C2, C3 = 32, 32, 64
R = B * K * M
NTOT = float(B * M * K)

F32 = jnp.float32
I32 = jnp.int32


_NS = 8


def _fps_body(xs_ref, ys_ref, zs_ref, cx_ref, cy_ref, cz_ref, dist_ref):
    sh = (B, _NS, N // _NS)
    lane = (lax.broadcasted_iota(I32, sh, 1) * (N // _NS)
            + lax.broadcasted_iota(I32, sh, 2))
    dist_ref[...] = jnp.full(sh, 1e10, F32)

    def body(i, far):
        xs = xs_ref[...]
        ys = ys_ref[...]
        zs = zs_ref[...]
        msk = lane == far[:, None, None]
        cx = jnp.sum(jnp.where(msk, xs, 0.0), axis=(1, 2))
        cy = jnp.sum(jnp.where(msk, ys, 0.0), axis=(1, 2))
        cz = jnp.sum(jnp.where(msk, zs, 0.0), axis=(1, 2))
        cx_ref[pl.ds(i, 1), :] = cx[None, :]
        cy_ref[pl.ds(i, 1), :] = cy[None, :]
        cz_ref[pl.ds(i, 1), :] = cz[None, :]
        dx = xs - cx[:, None, None]
        dy = ys - cy[:, None, None]
        dz = zs - cz[:, None, None]
        d = dx * dx + (dy * dy + dz * dz)
        dist = jnp.minimum(dist_ref[...], d)
        dist_ref[...] = dist
        mx = jnp.max(dist, axis=(1, 2), keepdims=True)
        far2 = jnp.min(jnp.where(dist == mx, lane, N), axis=(1, 2))
        return far2.astype(I32)

    lax.fori_loop(0, M, body, jnp.zeros((B,), I32))


_fps_call = pl.pallas_call(
    _fps_body,
    grid=(1,),
    in_specs=[pl.BlockSpec((B, _NS, N // _NS), lambda i: (0, 0, 0))] * 3,
    out_specs=[pl.BlockSpec((M, B), lambda i: (0, 0))] * 3,
    out_shape=[jax.ShapeDtypeStruct((M, B), F32)] * 3,
    scratch_shapes=[pltpu.VMEM((B, _NS, N // _NS), F32)],
)


MT = 512


def _knn_body(qx_ref, qy_ref, qz_ref, xs_ref, ys_ref, zs_ref, nn_ref, d2_ref):
    b = pl.program_id(0)
    qx = qx_ref[0, 0, 0, :]
    qy = qy_ref[0, 0, 0, :]
    qz = qz_ref[0, 0, 0, :]
    xs = xs_ref[0, 0, :]
    ys = ys_ref[0, 0, :]
    zs = zs_ref[0, 0, :]
    qq = qx * qx + qy * qy + qz * qz
    xx = xs * xs + ys * ys + zs * zs
    qxb = qx.astype(jnp.bfloat16).astype(F32)
    qyb = qy.astype(jnp.bfloat16).astype(F32)
    qzb = qz.astype(jnp.bfloat16).astype(F32)
    xsb = xs.astype(jnp.bfloat16).astype(F32)
    ysb = ys.astype(jnp.bfloat16).astype(F32)
    zsb = zs.astype(jnp.bfloat16).astype(F32)
    dot = (qxb[:, None] * xsb[None, :] + qyb[:, None] * ysb[None, :]
           + qzb[:, None] * zsb[None, :])
    d2_ref[...] = qq[:, None] + xx[None, :] - 2.0 * dot
    lane = lax.broadcasted_iota(I32, (MT, N), 1)

    def body(k, carry):
        d2 = d2_ref[...]
        m = jnp.min(d2, axis=1, keepdims=True)
        idx = jnp.min(jnp.where(d2 == m, lane, N), axis=1, keepdims=True)
        nn_ref[0, pl.ds(k, 1), :] = jnp.reshape(idx, (1, MT)) + b * N
        d2_ref[...] = jnp.where(lane == idx, jnp.inf, d2)
        return carry

    lax.fori_loop(0, K, body, 0)


_knn_call = pl.pallas_call(
    _knn_body,
    grid=(B, M // MT),
    in_specs=[pl.BlockSpec((1, 1, 1, MT), lambda b, t: (b, t, 0, 0))] * 3
    + [pl.BlockSpec((1, 1, N), lambda b, t: (b, 0, 0))] * 3,
    out_specs=pl.BlockSpec((1, K, MT), lambda b, t: (b, 0, t)),
    out_shape=jax.ShapeDtypeStruct((B, K, M), I32),
    scratch_shapes=[pltpu.VMEM((MT, N), F32)],
)


def _pre_body(x_ref, q_ref, w_ref, a_ref, c_ref):
    a_ref[0] = jnp.dot(x_ref[0], w_ref[...], preferred_element_type=F32)
    c_ref[0] = jnp.dot(q_ref[0], w_ref[0:8, :], preferred_element_type=F32)


_pre_call = pl.pallas_call(
    _pre_body,
    grid=(B,),
    in_specs=[
        pl.BlockSpec((1, N, 24), lambda b: (b, 0, 0)),
        pl.BlockSpec((1, M, 8), lambda b: (b, 0, 0)),
        pl.BlockSpec((24, C1), lambda b: (0, 0)),
    ],
    out_specs=[
        pl.BlockSpec((1, N, C1), lambda b: (b, 0, 0)),
        pl.BlockSpec((1, M, C1), lambda b: (b, 0, 0)),
    ],
    out_shape=[
        jax.ShapeDtypeStruct((B, N, C1), F32),
        jax.ShapeDtypeStruct((B, M, C1), F32),
    ],
)


_SC_CH = 128
_SC_NW = 32
_SC_PER_W = R // _SC_NW


@functools.cache
def _get_sc_gather():
    mesh = plsc.VectorSubcoreMesh(core_axis_name="c", subcore_axis_name="s")

    @functools.partial(
        pl.kernel,
        out_type=jax.ShapeDtypeStruct((R, C1), F32),
        mesh=mesh,
        scratch_types=[
            pltpu.VMEM((_SC_CH,), I32),
            pltpu.VMEM((_SC_CH, C1), F32),
            pltpu.SemaphoreType.DMA,
        ],
        compiler_params=pltpu.CompilerParams(use_tc_tiling_on_sc=False),
    )
    def _sc_gather(table_hbm, idx_hbm, out_hbm, idx_v, rows_v, sem):
        wid = lax.axis_index("s") * 2 + lax.axis_index("c")
        base = wid * _SC_PER_W

        def chunk(i, carry):
            off = pl.multiple_of(base + i * _SC_CH, _SC_CH)
            pltpu.sync_copy(idx_hbm.at[pl.ds(off, _SC_CH)], idx_v)
            pltpu.async_copy(table_hbm.at[idx_v], rows_v, sem).wait()
            pltpu.sync_copy(rows_v, out_hbm.at[pl.ds(off, _SC_CH)])
            return carry

        lax.fori_loop(0, _SC_PER_W // _SC_CH, chunk, 0)

    return _sc_gather


_GB = 8


def _stats_body(g_ref, c_ref, s_ref, q_ref):
    h = g_ref[...] - c_ref[...]
    ps = jnp.broadcast_to(jnp.sum(h, axis=(0, 1))[None, :], (8, C1))
    pq = jnp.broadcast_to(jnp.sum(h * h, axis=(0, 1))[None, :], (8, C1))

    @pl.when(pl.program_id(0) == 0)
    def _():
        s_ref[...] = jnp.zeros((8, C1), F32)
        q_ref[...] = jnp.zeros((8, C1), F32)

    s_ref[...] += ps
    q_ref[...] += pq


_stats_call = pl.pallas_call(
    _stats_body,
    grid=(B * K // _GB,),
    in_specs=[
        pl.BlockSpec((_GB, M, C1), lambda i: (i, 0, 0)),
        pl.BlockSpec((1, M, C1), lambda i: (i // (K // _GB), 0, 0)),
    ],
    out_specs=[pl.BlockSpec((8, C1), lambda i: (0, 0))] * 2,
    out_shape=[jax.ShapeDtypeStruct((8, C1), F32)] * 2,
)


def _bn_gelu(x, s_ref, q_ref, gamma_ref, beta_ref, ch):
    mean = s_ref[0, :] / NTOT
    var = q_ref[0, :] / NTOT - mean * mean
    inv = gamma_ref[0, :] / jnp.sqrt(var + 1e-5)
    x = (x - mean[None, :]) * inv[None, :] + beta_ref[0, :][None, :]
    return 0.5 * x * (1.0 + lax.erf(x * 0.7071067811865476))


def _layer_body(sub_c, chin, chout, *refs):
    if sub_c:
        g_ref, c_ref, s_ref, q_ref, gm_ref, bt_ref, w_ref, y_ref, s2_ref, q2_ref = refs
        h = g_ref[...] - c_ref[...]
    else:
        g_ref, s_ref, q_ref, gm_ref, bt_ref, w_ref, y_ref, s2_ref, q2_ref = refs
        h = g_ref[...]
    x = _bn_gelu(h, s_ref, q_ref, gm_ref, bt_ref, chin)
    y = jnp.dot(x.reshape(_GB * M, chin), w_ref[...],
                preferred_element_type=F32)
    y_ref[...] = y.reshape(_GB, M, chout)
    ps = jnp.broadcast_to(jnp.sum(y, axis=0)[None, :], (8, chout))
    pq = jnp.broadcast_to(jnp.sum(y * y, axis=0)[None, :], (8, chout))

    @pl.when(pl.program_id(0) == 0)
    def _():
        s2_ref[...] = jnp.zeros((8, chout), F32)
        q2_ref[...] = jnp.zeros((8, chout), F32)

    s2_ref[...] += ps
    q2_ref[...] += pq


def _make_layer_call(sub_c, chin, chout):
    in_specs = [pl.BlockSpec((_GB, M, chin), lambda i: (i, 0, 0))]
    if sub_c:
        in_specs.append(
            pl.BlockSpec((1, M, chin), lambda i: (i // (K // _GB), 0, 0)))
    in_specs += [
        pl.BlockSpec((8, chin), lambda i: (0, 0)),
        pl.BlockSpec((8, chin), lambda i: (0, 0)),
        pl.BlockSpec((1, chin), lambda i: (0, 0)),
        pl.BlockSpec((1, chin), lambda i: (0, 0)),
        pl.BlockSpec((chin, chout), lambda i: (0, 0)),
    ]
    return pl.pallas_call(
        functools.partial(_layer_body, sub_c, chin, chout),
        grid=(B * K // _GB,),
        in_specs=in_specs,
        out_specs=[
            pl.BlockSpec((_GB, M, chout), lambda i: (i, 0, 0)),
            pl.BlockSpec((8, chout), lambda i: (0, 0)),
            pl.BlockSpec((8, chout), lambda i: (0, 0)),
        ],
        out_shape=[
            jax.ShapeDtypeStruct((B * K, M, chout), F32),
            jax.ShapeDtypeStruct((8, chout), F32),
            jax.ShapeDtypeStruct((8, chout), F32),
        ],
    )


_layer2_call = _make_layer_call(True, C1, C2)
_layer3_call = _make_layer_call(False, C2, C3)


QT = 256


def _pool_body(y_ref, s_ref, q_ref, gm_ref, bt_ref, o_ref):
    x = _bn_gelu(y_ref[0], s_ref, q_ref, gm_ref, bt_ref, C3)
    o_ref[0] = jnp.max(x, axis=0)


_pool_call = pl.pallas_call(
    _pool_body,
    grid=(B, M // QT),
    in_specs=[
        pl.BlockSpec((1, K, QT, C3), lambda b, t: (b, 0, t, 0)),
        pl.BlockSpec((8, C3), lambda b, t: (0, 0)),
        pl.BlockSpec((8, C3), lambda b, t: (0, 0)),
        pl.BlockSpec((1, C3), lambda b, t: (0, 0)),
        pl.BlockSpec((1, C3), lambda b, t: (0, 0)),
    ],
    out_specs=pl.BlockSpec((1, QT, C3), lambda b, t: (b, t, 0)),
    out_shape=jax.ShapeDtypeStruct((B, M, C3), F32),
)


@jax.jit
def kernel(xyz, feat, W0, gamma0, beta0, W1, gamma1, beta1, W2, gamma2, beta2):
    xs = xyz[..., 0]
    ys = xyz[..., 1]
    zs = xyz[..., 2]
    fsh = (B, _NS, N // _NS)
    cxs, cys, czs = _fps_call(xs.reshape(fsh), ys.reshape(fsh), zs.reshape(fsh))
    new_xyz = jnp.stack([cxs, cys, czs], axis=-1).transpose(1, 0, 2)
    qshape = (B, M // MT, 1, MT)
    nn = _knn_call(
        cxs.T.reshape(qshape), cys.T.reshape(qshape), czs.T.reshape(qshape),
        xs.reshape(B, 1, N), ys.reshape(B, 1, N), zs.reshape(B, 1, N))

    Xp = jnp.pad(jnp.concatenate([xyz, feat], axis=-1), ((0, 0), (0, 0), (0, 5)))
    Qp = jnp.pad(new_xyz, ((0, 0), (0, 0), (0, 5)))
    W0p = jnp.pad(W0, ((0, 5), (0, 0)))
    A, C = _pre_call(Xp, Qp, W0p)

    G = _get_sc_gather()(A.reshape(B * N, C1), nn.reshape(-1))
    Gr = G.reshape(B * K, M, C1)

    s1, q1 = _stats_call(Gr, C)
    g0 = gamma0.reshape(1, C1)
    b0 = beta0.reshape(1, C1)
    y2, s2, q2 = _layer2_call(Gr, C, s1, q1, g0, b0, W1)
    y3, s3, q3 = _layer3_call(
        y2, s2, q2, gamma1.reshape(1, C2), beta1.reshape(1, C2), W2)
    h = _pool_call(
        y3.reshape(B, K, M, C3), s3, q3,
        gamma2.reshape(1, C3), beta2.reshape(1, C3))
    return new_xyz, h


# speedup vs baseline: 1.1338x; 1.0005x over previous
import functools

import jax
import jax.numpy as jnp
from jax import lax
from jax.experimental import pallas as pl
from jax.experimental.pallas import tpu as pltpu
from jax.experimental.pallas import tpu_sc as plsc

B, N, M, K, IN_CH = 4, 8192, 1024, 32, 16
CH0 = IN_CH + 3
C1, C2, C3 = 32, 32, 64
R = B * K * M
NTOT = float(B * M * K)

F32 = jnp.float32
I32 = jnp.int32


_NS = 16


def _fps_body(xs_ref, ys_ref, zs_ref, cx_ref, cy_ref, cz_ref, dist_ref):
    sh = (B, _NS, N // _NS)
    lane = (lax.broadcasted_iota(I32, sh, 1) * (N // _NS)
            + lax.broadcasted_iota(I32, sh, 2))
    dist_ref[...] = jnp.full(sh, 1e10, F32)

    def body(i, far):
        xs = xs_ref[...]
        ys = ys_ref[...]
        zs = zs_ref[...]
        msk = lane == far[:, None, None]
        cx = jnp.sum(jnp.where(msk, xs, 0.0), axis=(1, 2))
        cy = jnp.sum(jnp.where(msk, ys, 0.0), axis=(1, 2))
        cz = jnp.sum(jnp.where(msk, zs, 0.0), axis=(1, 2))
        cx_ref[pl.ds(i, 1), :] = cx[None, :]
        cy_ref[pl.ds(i, 1), :] = cy[None, :]
        cz_ref[pl.ds(i, 1), :] = cz[None, :]
        dx = xs - cx[:, None, None]
        dy = ys - cy[:, None, None]
        dz = zs - cz[:, None, None]
        d = dx * dx + (dy * dy + dz * dz)
        dist = jnp.minimum(dist_ref[...], d)
        dist_ref[...] = dist
        mx = jnp.max(dist, axis=(1, 2), keepdims=True)
        far2 = jnp.min(jnp.where(dist == mx, lane, N), axis=(1, 2))
        return far2.astype(I32)

    lax.fori_loop(0, M, body, jnp.zeros((B,), I32))


_fps_call = pl.pallas_call(
    _fps_body,
    grid=(1,),
    in_specs=[pl.BlockSpec((B, _NS, N // _NS), lambda i: (0, 0, 0))] * 3,
    out_specs=[pl.BlockSpec((M, B), lambda i: (0, 0))] * 3,
    out_shape=[jax.ShapeDtypeStruct((M, B), F32)] * 3,
    scratch_shapes=[pltpu.VMEM((B, _NS, N // _NS), F32)],
)


MT = 512


def _knn_body(qx_ref, qy_ref, qz_ref, xs_ref, ys_ref, zs_ref, nn_ref, d2_ref):
    b = pl.program_id(0)
    qx = qx_ref[0, 0, 0, :]
    qy = qy_ref[0, 0, 0, :]
    qz = qz_ref[0, 0, 0, :]
    xs = xs_ref[0, 0, :]
    ys = ys_ref[0, 0, :]
    zs = zs_ref[0, 0, :]
    qq = qx * qx + qy * qy + qz * qz
    xx = xs * xs + ys * ys + zs * zs
    qxb = qx.astype(jnp.bfloat16).astype(F32)
    qyb = qy.astype(jnp.bfloat16).astype(F32)
    qzb = qz.astype(jnp.bfloat16).astype(F32)
    xsb = xs.astype(jnp.bfloat16).astype(F32)
    ysb = ys.astype(jnp.bfloat16).astype(F32)
    zsb = zs.astype(jnp.bfloat16).astype(F32)
    dot = (qxb[:, None] * xsb[None, :] + qyb[:, None] * ysb[None, :]
           + qzb[:, None] * zsb[None, :])
    d2_ref[...] = qq[:, None] + xx[None, :] - 2.0 * dot
    lane = lax.broadcasted_iota(I32, (MT, N), 1)

    def body(k, carry):
        d2 = d2_ref[...]
        m = jnp.min(d2, axis=1, keepdims=True)
        idx = jnp.min(jnp.where(d2 == m, lane, N), axis=1, keepdims=True)
        nn_ref[0, pl.ds(k, 1), :] = jnp.reshape(idx, (1, MT)) + b * N
        d2_ref[...] = jnp.where(lane == idx, jnp.inf, d2)
        return carry

    lax.fori_loop(0, K, body, 0)


_knn_call = pl.pallas_call(
    _knn_body,
    grid=(B, M // MT),
    in_specs=[pl.BlockSpec((1, 1, 1, MT), lambda b, t: (b, t, 0, 0))] * 3
    + [pl.BlockSpec((1, 1, N), lambda b, t: (b, 0, 0))] * 3,
    out_specs=pl.BlockSpec((1, K, MT), lambda b, t: (b, 0, t)),
    out_shape=jax.ShapeDtypeStruct((B, K, M), I32),
    scratch_shapes=[pltpu.VMEM((MT, N), F32)],
)


def _pre_body(x_ref, q_ref, w_ref, a_ref, c_ref):
    a_ref[0] = jnp.dot(x_ref[0], w_ref[...], preferred_element_type=F32)
    c_ref[0] = jnp.dot(q_ref[0], w_ref[0:8, :], preferred_element_type=F32)


_pre_call = pl.pallas_call(
    _pre_body,
    grid=(B,),
    in_specs=[
        pl.BlockSpec((1, N, 24), lambda b: (b, 0, 0)),
        pl.BlockSpec((1, M, 8), lambda b: (b, 0, 0)),
        pl.BlockSpec((24, C1), lambda b: (0, 0)),
    ],
    out_specs=[
        pl.BlockSpec((1, N, C1), lambda b: (b, 0, 0)),
        pl.BlockSpec((1, M, C1), lambda b: (b, 0, 0)),
    ],
    out_shape=[
        jax.ShapeDtypeStruct((B, N, C1), F32),
        jax.ShapeDtypeStruct((B, M, C1), F32),
    ],
)


_SC_CH = 128
_SC_NW = 32
_SC_PER_W = R // _SC_NW


@functools.cache
def _get_sc_gather():
    mesh = plsc.VectorSubcoreMesh(core_axis_name="c", subcore_axis_name="s")

    @functools.partial(
        pl.kernel,
        out_type=jax.ShapeDtypeStruct((R, C1), F32),
        mesh=mesh,
        scratch_types=[
            pltpu.VMEM((_SC_CH,), I32),
            pltpu.VMEM((_SC_CH, C1), F32),
            pltpu.SemaphoreType.DMA,
        ],
        compiler_params=pltpu.CompilerParams(use_tc_tiling_on_sc=False),
    )
    def _sc_gather(table_hbm, idx_hbm, out_hbm, idx_v, rows_v, sem):
        wid = lax.axis_index("s") * 2 + lax.axis_index("c")
        base = wid * _SC_PER_W

        def chunk(i, carry):
            off = pl.multiple_of(base + i * _SC_CH, _SC_CH)
            pltpu.sync_copy(idx_hbm.at[pl.ds(off, _SC_CH)], idx_v)
            pltpu.async_copy(table_hbm.at[idx_v], rows_v, sem).wait()
            pltpu.sync_copy(rows_v, out_hbm.at[pl.ds(off, _SC_CH)])
            return carry

        lax.fori_loop(0, _SC_PER_W // _SC_CH, chunk, 0)

    return _sc_gather


_GB = 8


def _stats_body(g_ref, c_ref, s_ref, q_ref):
    h = g_ref[...] - c_ref[...]
    ps = jnp.broadcast_to(jnp.sum(h, axis=(0, 1))[None, :], (8, C1))
    pq = jnp.broadcast_to(jnp.sum(h * h, axis=(0, 1))[None, :], (8, C1))

    @pl.when(pl.program_id(0) == 0)
    def _():
        s_ref[...] = jnp.zeros((8, C1), F32)
        q_ref[...] = jnp.zeros((8, C1), F32)

    s_ref[...] += ps
    q_ref[...] += pq


_stats_call = pl.pallas_call(
    _stats_body,
    grid=(B * K // _GB,),
    in_specs=[
        pl.BlockSpec((_GB, M, C1), lambda i: (i, 0, 0)),
        pl.BlockSpec((1, M, C1), lambda i: (i // (K // _GB), 0, 0)),
    ],
    out_specs=[pl.BlockSpec((8, C1), lambda i: (0, 0))] * 2,
    out_shape=[jax.ShapeDtypeStruct((8, C1), F32)] * 2,
)


def _bn_gelu(x, s_ref, q_ref, gamma_ref, beta_ref, ch):
    mean = s_ref[0, :] / NTOT
    var = q_ref[0, :] / NTOT - mean * mean
    inv = gamma_ref[0, :] / jnp.sqrt(var + 1e-5)
    x = (x - mean[None, :]) * inv[None, :] + beta_ref[0, :][None, :]
    return 0.5 * x * (1.0 + lax.erf(x * 0.7071067811865476))


def _layer_body(sub_c, chin, chout, *refs):
    if sub_c:
        g_ref, c_ref, s_ref, q_ref, gm_ref, bt_ref, w_ref, y_ref, s2_ref, q2_ref = refs
        h = g_ref[...] - c_ref[...]
    else:
        g_ref, s_ref, q_ref, gm_ref, bt_ref, w_ref, y_ref, s2_ref, q2_ref = refs
        h = g_ref[...]
    x = _bn_gelu(h, s_ref, q_ref, gm_ref, bt_ref, chin)
    y = jnp.dot(x.reshape(_GB * M, chin), w_ref[...],
                preferred_element_type=F32)
    y_ref[...] = y.reshape(_GB, M, chout)
    ps = jnp.broadcast_to(jnp.sum(y, axis=0)[None, :], (8, chout))
    pq = jnp.broadcast_to(jnp.sum(y * y, axis=0)[None, :], (8, chout))

    @pl.when(pl.program_id(0) == 0)
    def _():
        s2_ref[...] = jnp.zeros((8, chout), F32)
        q2_ref[...] = jnp.zeros((8, chout), F32)

    s2_ref[...] += ps
    q2_ref[...] += pq


def _make_layer_call(sub_c, chin, chout):
    in_specs = [pl.BlockSpec((_GB, M, chin), lambda i: (i, 0, 0))]
    if sub_c:
        in_specs.append(
            pl.BlockSpec((1, M, chin), lambda i: (i // (K // _GB), 0, 0)))
    in_specs += [
        pl.BlockSpec((8, chin), lambda i: (0, 0)),
        pl.BlockSpec((8, chin), lambda i: (0, 0)),
        pl.BlockSpec((1, chin), lambda i: (0, 0)),
        pl.BlockSpec((1, chin), lambda i: (0, 0)),
        pl.BlockSpec((chin, chout), lambda i: (0, 0)),
    ]
    return pl.pallas_call(
        functools.partial(_layer_body, sub_c, chin, chout),
        grid=(B * K // _GB,),
        in_specs=in_specs,
        out_specs=[
            pl.BlockSpec((_GB, M, chout), lambda i: (i, 0, 0)),
            pl.BlockSpec((8, chout), lambda i: (0, 0)),
            pl.BlockSpec((8, chout), lambda i: (0, 0)),
        ],
        out_shape=[
            jax.ShapeDtypeStruct((B * K, M, chout), F32),
            jax.ShapeDtypeStruct((8, chout), F32),
            jax.ShapeDtypeStruct((8, chout), F32),
        ],
    )


_layer2_call = _make_layer_call(True, C1, C2)
_layer3_call = _make_layer_call(False, C2, C3)


QT = 256


def _pool_body(y_ref, s_ref, q_ref, gm_ref, bt_ref, o_ref):
    x = _bn_gelu(y_ref[0], s_ref, q_ref, gm_ref, bt_ref, C3)
    o_ref[0] = jnp.max(x, axis=0)


_pool_call = pl.pallas_call(
    _pool_body,
    grid=(B, M // QT),
    in_specs=[
        pl.BlockSpec((1, K, QT, C3), lambda b, t: (b, 0, t, 0)),
        pl.BlockSpec((8, C3), lambda b, t: (0, 0)),
        pl.BlockSpec((8, C3), lambda b, t: (0, 0)),
        pl.BlockSpec((1, C3), lambda b, t: (0, 0)),
        pl.BlockSpec((1, C3), lambda b, t: (0, 0)),
    ],
    out_specs=pl.BlockSpec((1, QT, C3), lambda b, t: (b, t, 0)),
    out_shape=jax.ShapeDtypeStruct((B, M, C3), F32),
)


@jax.jit
def kernel(xyz, feat, W0, gamma0, beta0, W1, gamma1, beta1, W2, gamma2, beta2):
    xs = xyz[..., 0]
    ys = xyz[..., 1]
    zs = xyz[..., 2]
    fsh = (B, _NS, N // _NS)
    cxs, cys, czs = _fps_call(xs.reshape(fsh), ys.reshape(fsh), zs.reshape(fsh))
    new_xyz = jnp.stack([cxs, cys, czs], axis=-1).transpose(1, 0, 2)
    qshape = (B, M // MT, 1, MT)
    nn = _knn_call(
        cxs.T.reshape(qshape), cys.T.reshape(qshape), czs.T.reshape(qshape),
        xs.reshape(B, 1, N), ys.reshape(B, 1, N), zs.reshape(B, 1, N))

    Xp = jnp.pad(jnp.concatenate([xyz, feat], axis=-1), ((0, 0), (0, 0), (0, 5)))
    Qp = jnp.pad(new_xyz, ((0, 0), (0, 0), (0, 5)))
    W0p = jnp.pad(W0, ((0, 5), (0, 0)))
    A, C = _pre_call(Xp, Qp, W0p)

    G = _get_sc_gather()(A.reshape(B * N, C1), nn.reshape(-1))
    Gr = G.reshape(B * K, M, C1)

    s1, q1 = _stats_call(Gr, C)
    g0 = gamma0.reshape(1, C1)
    b0 = beta0.reshape(1, C1)
    y2, s2, q2 = _layer2_call(Gr, C, s1, q1, g0, b0, W1)
    y3, s3, q3 = _layer3_call(
        y2, s2, q2, gamma1.reshape(1, C2), beta1.reshape(1, C2), W2)
    h = _pool_call(
        y3.reshape(B, K, M, C3), s3, q3,
        gamma2.reshape(1, C3), beta2.reshape(1, C3))
    return new_xyz, h
